# Initial kernel scaffold; baseline (speedup 1.0000x reference)
import jax
import jax.numpy as jnp
from jax.experimental import pallas as pl


def kernel(inputs, embeddings):
    raise NotImplementedError("write your pallas kernel here")


# speedup vs baseline: 4.8074x; 4.8074x over previous
import functools

import jax
import jax.numpy as jnp
from jax import lax
from jax.experimental import pallas as pl
from jax.experimental.pallas import tpu as pltpu
from jax.experimental.pallas import tpu_sc as plsc

MAX_POSITIONS = 1000000
EMBED_DIM = 32
BATCH = 16384
HIST = 200

N = BATCH * HIST
NW = 32
PER_W = N // NW
IDX_W = 128
CHUNK = 1024
ROWS_PER_CHUNK = CHUNK // IDX_W
CHUNKS_PER_W = PER_W // CHUNK


def _sc_gather(idx_hbm, table_hbm, out_hbm, idx_v, rows_v, sem):
    wid = lax.axis_index("s") * 2 + lax.axis_index("c")
    row_base = wid * (PER_W // IDX_W)
    out_base = wid * PER_W

    def body(c, _):
        r0 = row_base + c * ROWS_PER_CHUNK
        pltpu.sync_copy(idx_hbm.at[pl.ds(r0, ROWS_PER_CHUNK)], idx_v)
        copies = []
        for j in range(ROWS_PER_CHUNK):
            copies.append(
                pltpu.async_copy(
                    table_hbm.at[idx_v.at[j]],
                    rows_v.at[pl.ds(j * IDX_W, IDX_W)],
                    sem,
                )
            )
        for cp in copies:
            cp.wait()
        pltpu.sync_copy(
            rows_v, out_hbm.at[pl.ds(out_base + c * CHUNK, CHUNK)]
        )
        return _

    lax.fori_loop(0, CHUNKS_PER_W, body, 0)


@jax.jit
def _lookup(idx2d, table):
    mesh = plsc.VectorSubcoreMesh(core_axis_name="c", subcore_axis_name="s")
    f = pl.kernel(
        _sc_gather,
        out_type=jax.ShapeDtypeStruct((N, EMBED_DIM), jnp.float32),
        mesh=mesh,
        scratch_types=[
            pltpu.VMEM((ROWS_PER_CHUNK, IDX_W), jnp.int32),
            pltpu.VMEM((CHUNK, EMBED_DIM), jnp.float32),
            pltpu.SemaphoreType.DMA,
        ],
        compiler_params=pltpu.CompilerParams(use_tc_tiling_on_sc=False),
    )
    return f(idx2d, table)


def kernel(inputs, embeddings):
    idx2d = inputs.astype(jnp.int32).reshape(N // IDX_W, IDX_W)
    out = _lookup(idx2d, embeddings)
    return out.reshape(BATCH, HIST, EMBED_DIM)


# speedup vs baseline: 5.0338x; 1.0471x over previous
import jax
import jax.numpy as jnp
from jax import lax
from jax.experimental import pallas as pl
from jax.experimental.pallas import tpu as pltpu
from jax.experimental.pallas import tpu_sc as plsc

MAX_POSITIONS = 1000000
EMBED_DIM = 32
BATCH = 16384
HIST = 200

N = BATCH * HIST
NW = 32
PER_W = N // NW
IDX_W = 128
CHUNK = 1024
ROWS_PER_CHUNK = CHUNK // IDX_W
CHUNKS_PER_W = PER_W // CHUNK
NBUF = 2


def _sc_gather(idx_hbm, table_hbm, out_hbm, idx_v0, idx_v1, rows_v0,
               rows_v1, isem0, isem1, gsem0, gsem1, wsem0, wsem1):
    idx_v = [idx_v0, idx_v1]
    rows_v = [rows_v0, rows_v1]
    isem = [isem0, isem1]
    gsem = [gsem0, gsem1]
    wsem = [wsem0, wsem1]

    wid = lax.axis_index("s") * 2 + lax.axis_index("c")
    row_base = wid * (PER_W // IDX_W)
    out_base = wid * PER_W

    def start_idx(c, b):
        pltpu.async_copy(
            idx_hbm.at[pl.ds(row_base + c * ROWS_PER_CHUNK, ROWS_PER_CHUNK)],
            idx_v[b], isem[b])

    def wait_idx(b):
        pltpu.make_async_copy(
            idx_hbm.at[pl.ds(row_base, ROWS_PER_CHUNK)],
            idx_v[b], isem[b]).wait()

    def wait_write(b):
        pltpu.make_async_copy(
            rows_v[b], out_hbm.at[pl.ds(out_base, CHUNK)], wsem[b]).wait()

    def step(c, b, first, fetch_next):
        if not first:
            wait_write(b)
        wait_idx(b)
        for j in range(ROWS_PER_CHUNK):
            pltpu.async_copy(
                table_hbm.at[idx_v[b].at[j]],
                rows_v[b].at[pl.ds(j * IDX_W, IDX_W)], gsem[b])
        for j in range(ROWS_PER_CHUNK):
            pltpu.make_async_copy(
                table_hbm.at[idx_v[b].at[j]],
                rows_v[b].at[pl.ds(j * IDX_W, IDX_W)], gsem[b]).wait()
        if fetch_next:
            start_idx(c + NBUF, b)
        pltpu.async_copy(
            rows_v[b], out_hbm.at[pl.ds(out_base + c * CHUNK, CHUNK)],
            wsem[b])

    for b in range(NBUF):
        start_idx(b, b)
    for b in range(NBUF):
        step(b, b, first=True, fetch_next=True)

    def body(g, carry):
        for b in range(NBUF):
            step(g * NBUF + b, b, first=False, fetch_next=True)
        return carry

    lax.fori_loop(1, CHUNKS_PER_W // NBUF - 1, body, 0)

    for b in range(NBUF):
        step(CHUNKS_PER_W - NBUF + b, b, first=False, fetch_next=False)
    for b in range(NBUF):
        wait_write(b)


@jax.jit
def _lookup(idx2d, table):
    mesh = plsc.VectorSubcoreMesh(core_axis_name="c", subcore_axis_name="s")
    f = pl.kernel(
        _sc_gather,
        out_type=jax.ShapeDtypeStruct((N, EMBED_DIM), jnp.float32),
        mesh=mesh,
        scratch_types=[
            pltpu.VMEM((ROWS_PER_CHUNK, IDX_W), jnp.int32),
            pltpu.VMEM((ROWS_PER_CHUNK, IDX_W), jnp.int32),
            pltpu.VMEM((CHUNK, EMBED_DIM), jnp.float32),
            pltpu.VMEM((CHUNK, EMBED_DIM), jnp.float32),
            pltpu.SemaphoreType.DMA,
            pltpu.SemaphoreType.DMA,
            pltpu.SemaphoreType.DMA,
            pltpu.SemaphoreType.DMA,
            pltpu.SemaphoreType.DMA,
            pltpu.SemaphoreType.DMA,
        ],
        compiler_params=pltpu.CompilerParams(use_tc_tiling_on_sc=False),
    )
    return f(idx2d, table)


def kernel(inputs, embeddings):
    idx2d = inputs.astype(jnp.int32).reshape(N // IDX_W, IDX_W)
    out = _lookup(idx2d, embeddings)
    return out.reshape(BATCH, HIST, EMBED_DIM)
